# baseline (speedup 1.0000x reference)
import functools

import jax
import jax.numpy as jnp
from jax import lax
from jax.experimental import pallas as pl
from jax.experimental.pallas import tpu as pltpu
from jax.experimental.pallas import tpu_sc as plsc

NC = 2
NS = 16
NW = NC * NS
CH = 128
NBUF = 4


def _flat_gather(weight, idx3, B, D, cpw):
    rounds = cpw // NBUF
    mesh = plsc.VectorSubcoreMesh(
        core_axis_name="c", subcore_axis_name="s", num_cores=NC, num_subcores=NS
    )

    @functools.partial(
        pl.kernel,
        mesh=mesh,
        out_type=jax.ShapeDtypeStruct((B, D), jnp.float32),
        scratch_types=[
            pltpu.VMEM((cpw, CH), jnp.int32),
            pltpu.VMEM((NBUF, CH, D), jnp.float32),
            pltpu.SemaphoreType.DMA((NBUF,)),
        ],
        compiler_params=pltpu.CompilerParams(use_tc_tiling_on_sc=False),
    )
    def k(table_hbm, idx_hbm, out_hbm, idx_v, bufs, gsem):
        wid = lax.axis_index("s") * NC + lax.axis_index("c")
        pltpu.sync_copy(idx_hbm.at[wid], idx_v)
        base = wid * cpw

        for b in range(NBUF):
            pltpu.make_async_copy(
                table_hbm.at[idx_v.at[b]], bufs.at[b], gsem.at[b]
            ).start()

        def round_body(r, carry):
            for b in range(NBUF):
                j = r * NBUF + b
                pltpu.make_async_copy(
                    table_hbm.at[idx_v.at[j]], bufs.at[b], gsem.at[b]
                ).wait()
                pltpu.sync_copy(bufs.at[b], out_hbm.at[pl.ds((base + j) * CH, CH)])
                pltpu.make_async_copy(
                    table_hbm.at[idx_v.at[j + NBUF]], bufs.at[b], gsem.at[b]
                ).start()
            return carry

        lax.fori_loop(0, rounds - 1, round_body, 0)

        for b in range(NBUF):
            j = (rounds - 1) * NBUF + b
            pltpu.make_async_copy(
                table_hbm.at[idx_v.at[j]], bufs.at[b], gsem.at[b]
            ).wait()
            pltpu.sync_copy(bufs.at[b], out_hbm.at[pl.ds((base + j) * CH, CH)])

    return k(weight, idx3)


def kernel(indices, weight):
    B = indices.size
    D = weight.shape[1]
    cpw = B // (NW * CH)
    idx3 = indices.reshape(NW, cpw, CH).astype(jnp.int32)
    out = _flat_gather(weight, idx3, B, D, cpw)
    return out.reshape(indices.shape + (D,))
